# Initial kernel scaffold; baseline (speedup 1.0000x reference)
import jax
import jax.numpy as jnp
from jax.experimental import pallas as pl


def kernel(gate_logits):
    raise NotImplementedError("write your pallas kernel here")


# speedup vs baseline: 2.8919x; 2.8919x over previous
import jax
import jax.numpy as jnp
from jax.experimental import pallas as pl
from jax.experimental.pallas import tpu as pltpu

_TOKENS = 32768
_EXPERTS = 64
_TOPK = 8
_BT = 2048
_NB = _TOKENS // _BT


def _body(x_ref, out_ref, psum_ref, mask_ref):
    i = pl.program_id(0)
    x = x_ref[...]

    mx0 = jnp.max(x, axis=1, keepdims=True)
    e = jnp.exp(x - mx0)
    s = jnp.sum(e, axis=1, keepdims=True)
    p = e / s
    psum_part = jnp.sum(p, axis=0, keepdims=True)

    neg = jnp.float32(-jnp.inf)
    w = x
    mx = mx0
    for _ in range(_TOPK):
        w = jnp.where(w >= mx, neg, w)
        mx = jnp.max(w, axis=1, keepdims=True)
    sel = (x > mx).astype(jnp.float32)
    mask_part = jnp.max(sel, axis=0, keepdims=True)

    @pl.when(i == 0)
    def _init():
        psum_ref[...] = psum_part
        mask_ref[...] = mask_part

    @pl.when(i > 0)
    def _acc():
        psum_ref[...] = psum_ref[...] + psum_part
        mask_ref[...] = jnp.maximum(mask_ref[...], mask_part)

    @pl.when(i == _NB - 1)
    def _fin():
        t = jnp.sum(psum_ref[...]) * jnp.float32(1.0 / _TOKENS)
        msum = jnp.sum(mask_ref[...])
        out_ref[...] = jnp.full((1, 1), msum * t, jnp.float32)


def kernel(gate_logits):
    out = pl.pallas_call(
        _body,
        grid=(_NB,),
        in_specs=[pl.BlockSpec((_BT, _EXPERTS), lambda i: (i, 0))],
        out_specs=pl.BlockSpec((1, 1), lambda i: (0, 0)),
        out_shape=jax.ShapeDtypeStruct((1, 1), jnp.float32),
        scratch_shapes=[
            pltpu.VMEM((1, _EXPERTS), jnp.float32),
            pltpu.VMEM((1, _EXPERTS), jnp.float32),
        ],
    )(gate_logits)
    return out[0, 0]
